# baseline (speedup 1.0000x reference)
import functools

import jax
import jax.numpy as jnp
from jax.experimental import pallas as pl
from jax.experimental.pallas import tpu as pltpu
from jax.experimental.pallas import tpu_sc as plsc


def _tc_project_table(tok_embs, w, b):
    vocab, dim = tok_embs.shape
    half = vocab // 2
    blk = 10000
    nblocks = half // blk
    assert half % blk == 0

    def proj_kernel(lo_ref, hi_ref, w_ref, b_ref, o_ref):
        dn = (((1,), (1,)), ((), ()))
        o_ref[:, :dim] = (
            jax.lax.dot_general(
                lo_ref[...], w_ref[...], dn, preferred_element_type=jnp.float32
            )
            + b_ref[...]
        )
        o_ref[:, dim:] = (
            jax.lax.dot_general(
                hi_ref[...], w_ref[...], dn, preferred_element_type=jnp.float32
            )
            + b_ref[...]
        )

    return pl.pallas_call(
        proj_kernel,
        grid=(nblocks,),
        in_specs=[
            pl.BlockSpec((blk, dim), lambda i: (i, 0)),
            pl.BlockSpec((blk, dim), lambda i: (i + nblocks, 0)),
            pl.BlockSpec((dim, dim), lambda i: (0, 0)),
            pl.BlockSpec((1, dim), lambda i: (0, 0)),
        ],
        out_specs=pl.BlockSpec((blk, 2 * dim), lambda i: (i, 0)),
        out_shape=jax.ShapeDtypeStruct((half, 2 * dim), jnp.float32),
    )(tok_embs, tok_embs, w, b.reshape(1, dim))


def _sc_gather(table_fused, idx_mod):
    n = idx_mod.shape[0]
    width = table_fused.shape[1]
    window = 256
    assert n % window == 0
    mesh = plsc.VectorSubcoreMesh(core_axis_name="core", subcore_axis_name="subcore")
    idx2d = idx_mod.reshape(1, n)

    @functools.partial(
        pl.kernel,
        out_type=jax.ShapeDtypeStruct((n, width), table_fused.dtype),
        mesh=mesh,
    )
    def gather_kernel(tab_hbm, i_hbm, o_hbm):
        def body(i_vmem, o_vmem):
            pltpu.sync_copy(tab_hbm.at[i_vmem.at[0]], o_vmem)

        pltpu.emit_pipeline(
            body,
            grid=(n // window,),
            in_specs=[pl.BlockSpec((1, window), lambda i: (0, i))],
            out_specs=[pl.BlockSpec((window, width), lambda i: (i, 0))],
            core_axis_name=("core", "subcore"),
            dimension_semantics=(pltpu.PARALLEL,),
        )(i_hbm, o_hbm)

    return gather_kernel(table_fused, idx2d)


def kernel(x, tok_embs, W, b):
    batch, seqlen = x.shape
    vocab, dim = tok_embs.shape
    half = vocab // 2
    idx = x.reshape(-1)
    proj = _tc_project_table(tok_embs, W, b)
    rows = _sc_gather(proj, jnp.where(idx < half, idx, idx - half))
    z = jnp.where((idx >= half)[:, None], rows[:, dim:], rows[:, :dim])
    return z.reshape(batch, seqlen, dim)
